# baseline (speedup 1.0000x reference)
import dataclasses
import functools

import jax
import jax.numpy as jnp
from jax import lax
from jax.experimental import pallas as pl
from jax.experimental.pallas import tpu as pltpu
from jax.experimental.pallas import tpu_sc as plsc

NC = 2
NS = 16
NW = NC * NS
L = 16
W = 128


def kernel(h, r, t, ent_re, ent_im, rel_re, rel_im):
    B = h.shape[0]
    D = ent_re.shape[1]
    assert B % NW == 0
    b_per_w = B // NW
    assert b_per_w % W == 0
    nwin = b_per_w // W
    nchunk = D // L

    h32 = h.astype(jnp.int32)
    r32 = r.astype(jnp.int32)
    t32 = t.astype(jnp.int32)

    mesh = plsc.VectorSubcoreMesh(core_axis_name="c", subcore_axis_name="s")
    cp = pltpu.CompilerParams()
    if "needs_layout_passes" in pltpu.CompilerParams.__dataclass_fields__:
        cp = dataclasses.replace(cp, needs_layout_passes=False)
    if "use_tc_tiling_on_sc" in pltpu.CompilerParams.__dataclass_fields__:
        cp = dataclasses.replace(cp, use_tc_tiling_on_sc=False)

    @functools.partial(
        pl.kernel,
        mesh=mesh,
        compiler_params=cp,
        out_type=jax.ShapeDtypeStruct((B,), jnp.float32),
        scratch_types=[
            pltpu.VMEM((b_per_w,), jnp.int32),
            pltpu.VMEM((b_per_w,), jnp.int32),
            pltpu.VMEM((b_per_w,), jnp.int32),
            pltpu.VMEM((2, W, D), jnp.float32),
            pltpu.VMEM((2, W, D), jnp.float32),
            pltpu.VMEM((2, W, D), jnp.float32),
            pltpu.VMEM((2, W, D), jnp.float32),
            pltpu.VMEM((2, W, D), jnp.float32),
            pltpu.VMEM((2, W, D), jnp.float32),
            pltpu.VMEM((b_per_w,), jnp.float32),
            pltpu.SemaphoreType.DMA((2,)),
        ],
    )
    def kge_kernel(h_hbm, r_hbm, t_hbm, ere_hbm, eim_hbm, rre_hbm, rim_hbm,
                   out_hbm, hidx, ridx, tidx, bhr, bhi, brr, bri, btr, bti,
                   outv, sems):
        wid = lax.axis_index("s") * NC + lax.axis_index("c")
        base = wid * b_per_w
        pltpu.sync_copy(h_hbm.at[pl.ds(base, b_per_w)], hidx)
        pltpu.sync_copy(r_hbm.at[pl.ds(base, b_per_w)], ridx)
        pltpu.sync_copy(t_hbm.at[pl.ds(base, b_per_w)], tidx)

        def start_gathers(g, slot):
            hs = hidx.at[pl.ds(g * W, W)]
            rs = ridx.at[pl.ds(g * W, W)]
            ts = tidx.at[pl.ds(g * W, W)]
            sem = sems.at[slot]
            pltpu.async_copy(ere_hbm.at[hs], bhr.at[slot], sem)
            pltpu.async_copy(eim_hbm.at[hs], bhi.at[slot], sem)
            pltpu.async_copy(rre_hbm.at[rs], brr.at[slot], sem)
            pltpu.async_copy(rim_hbm.at[rs], bri.at[slot], sem)
            pltpu.async_copy(ere_hbm.at[ts], btr.at[slot], sem)
            pltpu.async_copy(eim_hbm.at[ts], bti.at[slot], sem)

        def drain(slot):
            pltpu.make_async_copy(ere_hbm.at[hidx.at[pl.ds(0, W)]],
                                  bhr.at[slot], sems.at[slot]).wait()
            pltpu.make_async_copy(eim_hbm.at[hidx.at[pl.ds(0, W)]],
                                  bhi.at[slot], sems.at[slot]).wait()
            pltpu.make_async_copy(rre_hbm.at[ridx.at[pl.ds(0, W)]],
                                  brr.at[slot], sems.at[slot]).wait()
            pltpu.make_async_copy(rim_hbm.at[ridx.at[pl.ds(0, W)]],
                                  bri.at[slot], sems.at[slot]).wait()
            pltpu.make_async_copy(ere_hbm.at[tidx.at[pl.ds(0, W)]],
                                  btr.at[slot], sems.at[slot]).wait()
            pltpu.make_async_copy(eim_hbm.at[tidx.at[pl.ds(0, W)]],
                                  bti.at[slot], sems.at[slot]).wait()

        lane = lax.iota(jnp.int32, L)

        def compute(g, slot):
            @pl.loop(0, W // L)
            def _(grp):
                w0 = grp * L
                res = jnp.zeros((L,), jnp.float32)
                for j in range(L):
                    w = w0 + j
                    acc = jnp.zeros((L,), jnp.float32)
                    for c in range(nchunk):
                        sl = pl.ds(c * L, L)
                        hr = bhr[slot, w, sl]
                        hi = bhi[slot, w, sl]
                        rr = brr[slot, w, sl]
                        ri = bri[slot, w, sl]
                        tr = btr[slot, w, sl]
                        ti = bti[slot, w, sl]
                        acc = acc + (hr * rr - hi * ri) * tr \
                                  + (hr * ri + hi * rr) * ti
                    res = jnp.where(lane == j, jnp.sum(acc), res)
                outv[pl.ds(g * W + w0, L)] = res

        start_gathers(0, 0)
        for g in range(nwin):
            if g + 1 < nwin:
                start_gathers(g + 1, (g + 1) % 2)
            drain(g % 2)
            compute(g, g % 2)

        pltpu.sync_copy(outv, out_hbm.at[pl.ds(base, b_per_w)])

    return kge_kernel(h32, r32, t32, ent_re, ent_im, rel_re, rel_im)
